# Initial kernel scaffold; baseline (speedup 1.0000x reference)
import functools

import jax
import jax.numpy as jnp
from jax import lax
from jax.experimental import pallas as pl
from jax.experimental.pallas import tpu as pltpu
from jax.experimental.pallas import tpu_sc as plsc

NEG_SLOPE = 0.2

NC = 2
NS = 16
NW = NC * NS
B = 128


def _mm_body(x_ref, w_ref, ms_ref, md_ref, t64_ref, td_ref):
    h = jnp.dot(x_ref[...], w_ref[...], preferred_element_type=jnp.float32)
    ts = jnp.dot(h, ms_ref[...], preferred_element_type=jnp.float32)
    td = jnp.dot(h, md_ref[...], preferred_element_type=jnp.float32)
    t64_ref[...] = jnp.concatenate([h, ts], axis=1)
    td_ref[...] = td


def _fin_body(p_ref, b_ref, o_ref):
    s = p_ref[0] + p_ref[1]
    o_ref[...] = s[:, :32] / s[:, 32:] + b_ref[...]


def _make_sc_kernel(np_rows, ch):
    mesh = plsc.VectorSubcoreMesh(core_axis_name="c", subcore_axis_name="s")
    rpt = np_rows // NS

    @functools.partial(
        pl.kernel,
        out_type=jax.ShapeDtypeStruct((NC, np_rows, 64), jnp.float32),
        mesh=mesh,
        scratch_types=[
            pltpu.VMEM((B,), jnp.int32),
            pltpu.VMEM((B,), jnp.int32),
            pltpu.VMEM((B, 64), jnp.float32),
            pltpu.VMEM((B, 32), jnp.float32),
            pltpu.VMEM((B, 64), jnp.float32),
            pltpu.VMEM_SHARED((np_rows, 64), jnp.float32),
        ],
    )
    def sc_kernel(t64, t_d, srcp, dstp, zz, outp, idx_s, idx_d, rows64, rows_d,
                  outb, acc):
        cid = lax.axis_index("c")
        sid = lax.axis_index("s")
        wid = sid * NC + cid
        r0 = sid * rpt
        pltpu.sync_copy(zz.at[pl.ds(r0, rpt)], acc.at[pl.ds(r0, rpt)])
        plsc.subcore_barrier()

        def chunk(c, carry):
            base = (wid * ch + c) * B
            pltpu.sync_copy(srcp.at[pl.ds(base, B)], idx_s)
            pltpu.sync_copy(dstp.at[pl.ds(base, B)], idx_d)
            pltpu.sync_copy(t64.at[idx_s], rows64)
            pltpu.sync_copy(t_d.at[idx_d], rows_d)

            def edge(e, carry2):
                for q in (0, 16):
                    a_s = rows64[e, pl.ds(32 + q, 16)]
                    a_d = rows_d[e, pl.ds(q, 16)]
                    al = a_s + a_d
                    al = jnp.where(al >= 0.0, al, al * NEG_SLOPE)
                    ex = jnp.exp(al)
                    outb[e, pl.ds(q, 16)] = rows64[e, pl.ds(q, 16)] * ex
                    outb[e, pl.ds(32 + q, 16)] = ex
                return carry2

            lax.fori_loop(0, B, edge, 0)
            pltpu.sync_copy(outb, acc.at[idx_d], add=True)
            return carry

        lax.fori_loop(0, ch, chunk, 0)
        plsc.subcore_barrier()
        pltpu.sync_copy(acc.at[pl.ds(r0, rpt)], outp.at[cid, pl.ds(r0, rpt)])

    return sc_kernel


def kernel(x, edge_index, W, att_src, att_dst, bias):
    n, in_c = x.shape
    e = edge_index.shape[1]
    hc = W.shape[1]
    heads = att_src.shape[1]
    out_c = att_src.shape[2]

    kk = jnp.arange(hc)
    same_head = (kk[:, None] // out_c) == (kk[None, :] // out_c)
    a_s = att_src.reshape(heads, out_c)
    a_d = att_dst.reshape(heads, out_c)
    vals_s = a_s[kk[None, :] // out_c, kk[:, None] % out_c]
    vals_d = a_d[kk[None, :] // out_c, kk[:, None] % out_c]
    ms = jnp.where(same_head, vals_s, 0.0).astype(jnp.float32)
    md = jnp.where(same_head, vals_d, 0.0).astype(jnp.float32)

    np_rows = ((n + 16) + 639) // 640 * 640
    xp = jnp.pad(x.astype(jnp.float32), ((0, np_rows - n), (0, 0)))

    m = e + n
    ch = -(-m // (NW * B))
    m_pad = NW * B * ch
    loop = jnp.arange(n, dtype=edge_index.dtype)
    srcp = jnp.concatenate(
        [edge_index[0], loop,
         jnp.zeros((m_pad - m,), dtype=edge_index.dtype)])
    dstp = jnp.concatenate(
        [edge_index[1], loop,
         jnp.full((m_pad - m,), n, dtype=edge_index.dtype)])
    zz = jnp.zeros((np_rows, 64), jnp.float32)

    grid1 = np_rows // 640
    t64, t_d = pl.pallas_call(
        _mm_body,
        grid=(grid1,),
        in_specs=[
            pl.BlockSpec((640, in_c), lambda i: (i, 0)),
            pl.BlockSpec((in_c, hc), lambda i: (0, 0)),
            pl.BlockSpec((hc, hc), lambda i: (0, 0)),
            pl.BlockSpec((hc, hc), lambda i: (0, 0)),
        ],
        out_specs=[
            pl.BlockSpec((640, 64), lambda i: (i, 0)),
            pl.BlockSpec((640, 32), lambda i: (i, 0)),
        ],
        out_shape=[
            jax.ShapeDtypeStruct((np_rows, 64), jnp.float32),
            jax.ShapeDtypeStruct((np_rows, 32), jnp.float32),
        ],
    )(xp, W.astype(jnp.float32), ms, md)

    outp = _make_sc_kernel(np_rows, ch)(t64, t_d, srcp, dstp, zz)

    rows_blk = 500
    grid3 = n // rows_blk
    out = pl.pallas_call(
        _fin_body,
        grid=(grid3,),
        in_specs=[
            pl.BlockSpec((2, rows_blk, 64), lambda i: (0, i, 0)),
            pl.BlockSpec((1, hc), lambda i: (0, 0)),
        ],
        out_specs=pl.BlockSpec((rows_blk, hc), lambda i: (i, 0)),
        out_shape=jax.ShapeDtypeStruct((n, hc), jnp.float32),
    )(outp, bias.reshape(1, hc).astype(jnp.float32))
    return out


# speedup vs baseline: 48.4201x; 48.4201x over previous
import functools

import jax
import jax.numpy as jnp
from jax import lax
from jax.experimental import pallas as pl
from jax.experimental.pallas import tpu as pltpu
from jax.experimental.pallas import tpu_sc as plsc

NEG_SLOPE = 0.2

NC = 2
NS = 16
NW = NC * NS
B = 128


def _mm_body(x_ref, w_ref, ms_ref, md_ref, t64_ref, td_ref):
    h = jnp.dot(x_ref[...], w_ref[...], preferred_element_type=jnp.float32)
    ts = jnp.dot(h, ms_ref[...], preferred_element_type=jnp.float32)
    td = jnp.dot(h, md_ref[...], preferred_element_type=jnp.float32)
    t64_ref[...] = jnp.concatenate([h, ts], axis=1)
    td_ref[...] = td


def _fin_body(p_ref, b_ref, o_ref):
    s = p_ref[0] + p_ref[1]
    o_ref[...] = s[:, :32] / s[:, 32:] + b_ref[...]


def _make_sc_kernel(np_rows, ch):
    mesh = plsc.VectorSubcoreMesh(core_axis_name="c", subcore_axis_name="s")
    rpt = np_rows // NS

    @functools.partial(
        pl.kernel,
        out_type=jax.ShapeDtypeStruct((NC, np_rows, 64), jnp.float32),
        mesh=mesh,
        scratch_types=[
            pltpu.VMEM((B,), jnp.int32),
            pltpu.VMEM((B,), jnp.int32),
            pltpu.VMEM((B, 64), jnp.float32),
            pltpu.VMEM((B, 32), jnp.float32),
            pltpu.VMEM((B, 64), jnp.float32),
            pltpu.VMEM((np_rows // NS, 64), jnp.float32),
            pltpu.VMEM_SHARED((np_rows, 64), jnp.float32),
            pltpu.SemaphoreType.DMA,
            pltpu.SemaphoreType.DMA,
        ],
        compiler_params=pltpu.CompilerParams(use_tc_tiling_on_sc=False),
    )
    def sc_kernel(t64, t_d, srcp, dstp, zz, outp, idx_s, idx_d, rows64, rows_d,
                  outb, stage, acc, sem1, sem2):
        cid = lax.axis_index("c")
        sid = lax.axis_index("s")
        wid = sid * NC + cid
        r0 = sid * rpt
        pltpu.sync_copy(zz.at[pl.ds(r0, rpt)], stage)
        pltpu.sync_copy(stage, acc.at[pl.ds(r0, rpt)])
        plsc.subcore_barrier()

        def chunk(c, carry):
            base = (wid * ch + c) * B
            pltpu.sync_copy(srcp.at[pl.ds(base, B)], idx_s)
            pltpu.sync_copy(dstp.at[pl.ds(base, B)], idx_d)
            cp1 = pltpu.async_copy(t64.at[idx_s], rows64, sem1)
            cp2 = pltpu.async_copy(t_d.at[idx_d], rows_d, sem2)
            cp1.wait()
            cp2.wait()

            def edge(e, carry2):
                for q in (0, 16):
                    a_s = rows64[e, pl.ds(32 + q, 16)]
                    a_d = rows_d[e, pl.ds(q, 16)]
                    al = a_s + a_d
                    al = jnp.where(al >= 0.0, al, al * NEG_SLOPE)
                    ex = jnp.exp(al)
                    outb[e, pl.ds(q, 16)] = rows64[e, pl.ds(q, 16)] * ex
                    outb[e, pl.ds(32 + q, 16)] = ex
                return carry2

            lax.fori_loop(0, B, edge, 0)
            pltpu.sync_copy(outb, acc.at[idx_d], add=True)
            return carry

        lax.fori_loop(0, ch, chunk, 0)
        plsc.subcore_barrier()
        pltpu.sync_copy(acc.at[pl.ds(r0, rpt)], stage)
        pltpu.sync_copy(stage, outp.at[cid, pl.ds(r0, rpt)])

    return sc_kernel


def kernel(x, edge_index, W, att_src, att_dst, bias):
    n, in_c = x.shape
    e = edge_index.shape[1]
    hc = W.shape[1]
    heads = att_src.shape[1]
    out_c = att_src.shape[2]

    kk = jnp.arange(hc)
    same_head = (kk[:, None] // out_c) == (kk[None, :] // out_c)
    a_s = att_src.reshape(heads, out_c)
    a_d = att_dst.reshape(heads, out_c)
    vals_s = a_s[kk[None, :] // out_c, kk[:, None] % out_c]
    vals_d = a_d[kk[None, :] // out_c, kk[:, None] % out_c]
    ms = jnp.where(same_head, vals_s, 0.0).astype(jnp.float32)
    md = jnp.where(same_head, vals_d, 0.0).astype(jnp.float32)

    np_rows = ((n + 16) + 639) // 640 * 640
    xp = jnp.pad(x.astype(jnp.float32), ((0, np_rows - n), (0, 0)))

    m = e + n
    ch = -(-m // (NW * B))
    m_pad = NW * B * ch
    loop = jnp.arange(n, dtype=edge_index.dtype)
    srcp = jnp.concatenate(
        [edge_index[0], loop,
         jnp.zeros((m_pad - m,), dtype=edge_index.dtype)])
    dstp = jnp.concatenate(
        [edge_index[1], loop,
         jnp.full((m_pad - m,), n, dtype=edge_index.dtype)])
    zz = jnp.zeros((np_rows, 64), jnp.float32)

    grid1 = np_rows // 640
    t64, t_d = pl.pallas_call(
        _mm_body,
        grid=(grid1,),
        in_specs=[
            pl.BlockSpec((640, in_c), lambda i: (i, 0)),
            pl.BlockSpec((in_c, hc), lambda i: (0, 0)),
            pl.BlockSpec((hc, hc), lambda i: (0, 0)),
            pl.BlockSpec((hc, hc), lambda i: (0, 0)),
        ],
        out_specs=[
            pl.BlockSpec((640, 64), lambda i: (i, 0)),
            pl.BlockSpec((640, 32), lambda i: (i, 0)),
        ],
        out_shape=[
            jax.ShapeDtypeStruct((np_rows, 64), jnp.float32),
            jax.ShapeDtypeStruct((np_rows, 32), jnp.float32),
        ],
    )(xp, W.astype(jnp.float32), ms, md)

    outp = _make_sc_kernel(np_rows, ch)(t64, t_d, srcp, dstp, zz)

    rows_blk = 640
    grid3 = np_rows // rows_blk
    out = pl.pallas_call(
        _fin_body,
        grid=(grid3,),
        in_specs=[
            pl.BlockSpec((2, rows_blk, 64), lambda i: (0, i, 0)),
            pl.BlockSpec((1, hc), lambda i: (0, 0)),
        ],
        out_specs=pl.BlockSpec((rows_blk, hc), lambda i: (i, 0)),
        out_shape=jax.ShapeDtypeStruct((np_rows, hc), jnp.float32),
    )(outp, bias.reshape(1, hc).astype(jnp.float32))
    return out[:n]
